# baseline (speedup 1.0000x reference)
import functools

import jax
import jax.numpy as jnp
from jax import lax
from jax.experimental import pallas as pl
from jax.experimental.pallas import tpu as pltpu
from jax.experimental.pallas import tpu_sc as plsc

NC = 2
NS = 16
NW = NC * NS

B1 = 16384
SL = 50
D = 64
V = 1000000

CB = 8
RP = CB * SL
BW = B1 // NW
NCH = BW // CB

T_ROWS = 8000


def _repack_body(in_ref, out_ref):
    y = in_ref[...].reshape(in_ref.shape[0] // 2, 2, D)
    out_ref[...] = jnp.concatenate([y[:, 0, :], y[:, 1, :]], axis=-1)


def _table_pairs(embedds):
    return pl.pallas_call(
        _repack_body,
        out_shape=jax.ShapeDtypeStruct((V // 2, 2 * D), jnp.float32),
        grid=(V // T_ROWS,),
        in_specs=[pl.BlockSpec((T_ROWS, D), lambda i: (i, 0))],
        out_specs=pl.BlockSpec((T_ROWS // 2, 2 * D), lambda i: (i, 0)),
    )(embedds)


O_NB = 64


def _unpack_body(in_ref, out_ref):
    z = in_ref[...]
    n = z.shape[0]
    a = z[:, :D].reshape(n, 1, D)
    b = z[:, D:].reshape(n, 1, D)
    w = jnp.concatenate([a, b], axis=1).reshape(2 * n // SL, SL, D)
    out_ref[...] = w


def _unpack(out1):
    return pl.pallas_call(
        _unpack_body,
        out_shape=jax.ShapeDtypeStruct((B1, SL, D), jnp.float32),
        grid=(B1 // O_NB,),
        in_specs=[pl.BlockSpec((O_NB * SL // 2, 2 * D), lambda i: (i, 0))],
        out_specs=pl.BlockSpec((O_NB, SL, D), lambda i: (i, 0, 0)),
    )(out1)


def _sc_gather(table2, idx1d):
    mesh = plsc.VectorSubcoreMesh(
        core_axis_name="c", subcore_axis_name="s",
        num_cores=NC, num_subcores=NS)

    @functools.partial(
        pl.kernel,
        out_type=jax.ShapeDtypeStruct((B1 * SL // 2, 2 * D), jnp.float32),
        mesh=mesh,
        scratch_types=[
            pltpu.VMEM((RP,), jnp.int32),
            pltpu.VMEM((RP,), jnp.int32),
            pltpu.VMEM((RP,), jnp.int32),
            pltpu.VMEM((RP,), jnp.int32),
            pltpu.VMEM((RP,), jnp.int32),
            pltpu.VMEM((RP, 2 * D), jnp.float32),
            pltpu.VMEM((RP, 2 * D), jnp.float32),
            pltpu.VMEM((RP // 2, 2 * D), jnp.float32),
            pltpu.SemaphoreType.DMA,
            pltpu.SemaphoreType.DMA,
            pltpu.SemaphoreType.DMA,
            pltpu.SemaphoreType.DMA,
        ],
    )
    def k(table_hbm, idx_hbm, out_hbm, idx_v, widx_va, widx_vb,
          poff_va, poff_vb, g_va, g_vb, out_v, sem_i, sem_g0, sem_g1,
          sem_o):
        wid = lax.axis_index("s") * NC + lax.axis_index("c")
        rbase = wid * BW * SL
        obase = wid * BW * SL // 2
        widx_v = [widx_va, widx_vb]
        poff_v = [poff_va, poff_vb]
        g_v = [g_va, g_vb]
        sem_g = [sem_g0, sem_g1]

        def idx_copy(ci):
            return pltpu.make_async_copy(
                idx_hbm.at[pl.ds(rbase + ci * RP, RP)], idx_v, sem_i)

        def vcomp(p):
            def body(vi, c):
                v = idx_v[pl.ds(vi * 16, 16)]
                widx_v[p][pl.ds(vi * 16, 16)] = lax.shift_right_logical(v, 1)
                poff_v[p][pl.ds(vi * 16, 16)] = (v & 1) * D
                return c
            lax.fori_loop(0, RP // 16, body, 0)

        def fire_gather(p, sem):
            pltpu.async_copy(table_hbm.at[widx_v[p]], g_v[p], sem)

        def select(p):
            def body(g, c):
                poff16 = poff_v[p][pl.ds(g * 16, 16)]
                for u in range(16):
                    r = g * 16 + u
                    rp = g * 8 + u // 2
                    half = (u % 2) * D
                    off = poff16[u]
                    for q in range(D // 16):
                        out_v[rp, pl.ds(half + q * 16, 16)] = (
                            g_v[p][r, pl.ds(off + q * 16, 16)])
                return c
            lax.fori_loop(0, RP // 16, body, 0)

        def out_copy(ci):
            off = pl.multiple_of(obase + ci * (RP // 2), 8)
            return pltpu.make_async_copy(
                out_v, out_hbm.at[pl.ds(off, RP // 2)], sem_o)

        idx_copy(0).start()
        idx_copy(0).wait()
        vcomp(0)
        fire_gather(0, sem_g[0])
        idx_copy(1).start()

        def chunk2(ci2, carry):
            for b in range(2):
                ci = ci2 * 2 + b
                p, p1 = b, 1 - b

                @pl.when(ci + 1 < NCH)
                def _(ci=ci, p1=p1):
                    idx_copy(ci + 1).wait()
                    vcomp(p1)
                    fire_gather(p1, sem_g[p1])

                    @pl.when(ci + 2 < NCH)
                    def _(ci=ci):
                        idx_copy(ci + 2).start()

                pltpu.make_async_copy(
                    table_hbm.at[widx_v[p]], g_v[p], sem_g[p]).wait()

                @pl.when(ci > 0)
                def _(ci=ci):
                    out_copy(ci - 1).wait()

                select(p)
                out_copy(ci).start()
            return carry

        lax.fori_loop(0, NCH // 2, chunk2, 0)
        out_copy(NCH - 1).wait()

    return k(table2, idx1d)


@jax.jit
def _lookup(embedds, input):
    table2 = _table_pairs(embedds)
    idx1d = input.reshape(-1).astype(jnp.int32)
    out1 = _sc_gather(table2, idx1d)
    return _unpack(out1)


def kernel(embedds, input):
    return _lookup(embedds, input)
